# baseline (speedup 1.0000x reference)
import functools

import jax
import jax.numpy as jnp
from jax import lax
from jax.experimental import pallas as pl
from jax.experimental.pallas import tpu as pltpu
from jax.experimental.pallas import tpu_sc as plsc

N, E, D_IN, H, L, T, G, DE = 10000, 320000, 128, 128, 3, 12, 64, 16
NP = 10240
HALF = NP // 2
EPT = E // 16
C = 80
CHT = EPT // C
ROWS_PER_TILE = HALF // 16

_f32 = jnp.float32


def _embed_body(x_ref, w_ref, b_ref, o_ref):
    o_ref[...] = jnp.dot(x_ref[...], w_ref[...],
                         preferred_element_type=_f32) + b_ref[...]


def _tc_embed(x_pad, W_emb, b_emb2):
    blk = NP // 10
    return pl.pallas_call(
        _embed_body,
        grid=(10,),
        in_specs=[
            pl.BlockSpec((blk, D_IN), lambda i: (i, 0)),
            pl.BlockSpec((D_IN, H), lambda i: (0, 0)),
            pl.BlockSpec((1, H), lambda i: (0, 0)),
        ],
        out_specs=pl.BlockSpec((blk, H), lambda i: (i, 0)),
        out_shape=jax.ShapeDtypeStruct((NP, H), _f32),
    )(x_pad, W_emb, b_emb2)


def _edge_body(a_ref, w_ref, o_ref):
    o_ref[...] = jnp.dot(a_ref[...], w_ref[...], preferred_element_type=_f32)


def _tc_edge(edge_attr, edge_W):
    blk = E // 80
    return pl.pallas_call(
        _edge_body,
        grid=(80,),
        in_specs=[
            pl.BlockSpec((blk, DE), lambda i: (i, 0)),
            pl.BlockSpec((DE, H), lambda i: (0, 0)),
        ],
        out_specs=pl.BlockSpec((blk, H), lambda i: (i, 0)),
        out_shape=jax.ShapeDtypeStruct((E, H), _f32),
    )(edge_attr, edge_W)


def _update_body(agg_ref, w_ref, b_ref, h_ref, o_ref):
    u = jnp.dot(agg_ref[...], w_ref[...], preferred_element_type=_f32) + b_ref[...]
    o_ref[...] = jnp.maximum(u, 0.0) + h_ref[...]


def _tc_update(agg, W, b2, h):
    blk = NP // 10
    return pl.pallas_call(
        _update_body,
        grid=(10,),
        in_specs=[
            pl.BlockSpec((blk, H), lambda i: (i, 0)),
            pl.BlockSpec((H, H), lambda i: (0, 0)),
            pl.BlockSpec((1, H), lambda i: (0, 0)),
            pl.BlockSpec((blk, H), lambda i: (i, 0)),
        ],
        out_specs=pl.BlockSpec((blk, H), lambda i: (i, 0)),
        out_shape=jax.ShapeDtypeStruct((NP, H), _f32),
    )(agg, W, b2, h)


def _pool_head_body(h_ref, batch_ref, wh_ref, bh_ref, o_ref):
    onehot = (batch_ref[...] ==
              lax.broadcasted_iota(jnp.int32, (NP, G), 1)).astype(_f32)
    graph_repr = lax.dot_general(onehot, h_ref[...],
                                 (((0,), (0,)), ((), ())),
                                 preferred_element_type=_f32)
    o_ref[...] = jnp.dot(graph_repr, wh_ref[...],
                         preferred_element_type=_f32) + bh_ref[...]


def _tc_pool_head(h, batch_col, W_head, b_head2):
    return pl.pallas_call(
        _pool_head_body,
        in_specs=[
            pl.BlockSpec((NP, H), lambda: (0, 0)),
            pl.BlockSpec((NP, 1), lambda: (0, 0)),
            pl.BlockSpec((H, T), lambda: (0, 0)),
            pl.BlockSpec((1, T), lambda: (0, 0)),
        ],
        out_specs=pl.BlockSpec((G, T), lambda: (0, 0)),
        out_shape=jax.ShapeDtypeStruct((G, T), _f32),
    )(h, batch_col, W_head, b_head2)


def _sc_body(h_hbm, e_hbm, pk_hbm, ei_hbm, jm_hbm, out_hbm,
             pk_v, ei_v, src_b, dst_b, jm_v, e_buf, acc, sem):
    cid = lax.axis_index("c")
    sid = lax.axis_index("s")
    wid = cid * 16 + sid

    pltpu.sync_copy(pk_hbm.at[wid], pk_v)
    pltpu.sync_copy(ei_hbm.at[wid], ei_v)
    pltpu.sync_copy(jm_hbm.at[wid], jm_v)

    def zrow(r, carry):
        for q in range(H // 16):
            e_buf[r, pl.ds(q * 16, 16)] = jnp.zeros((16,), _f32)
        return carry

    lax.fori_loop(0, C, zrow, 0)
    for k in range(ROWS_PER_TILE // C):
        pltpu.sync_copy(e_buf, acc.at[pl.ds(sid * ROWS_PER_TILE + k * C, C)])

    plsc.subcore_barrier()

    def chunk(j, carry):
        for q in range(C // 16):
            sl = pl.ds(q * 16, 16)
            v = pk_v[j, sl]
            neg = v < 0
            src_b[sl] = jnp.where(neg, -1, lax.bitwise_and(v, 0x3FFF))
            dst_b[sl] = jnp.where(neg, -1, lax.shift_right_logical(v, 14))
        pltpu.sync_copy(
            e_hbm.at[plsc.Indices(ei_v.at[j], ignored_value=-1)], e_buf)
        pltpu.sync_copy(
            h_hbm.at[plsc.Indices(src_b, ignored_value=-1)],
            e_buf, add=True)

        def row(r, carry2):
            for q in range(H // 16):
                sl = pl.ds(q * 16, 16)
                e_buf[r, sl] = jnp.maximum(e_buf[r, sl], 0.0)
            return carry2

        lax.fori_loop(0, C, row, 0)
        pltpu.sync_copy(
            e_buf, acc.at[plsc.Indices(dst_b, ignored_value=-1)],
            add=True)
        return carry

    jm = jm_v[pl.ds(0, 16)][0]
    lax.fori_loop(0, jm, chunk, 0)
    plsc.subcore_barrier()
    rows = pl.ds(sid * ROWS_PER_TILE, ROWS_PER_TILE)
    pltpu.sync_copy(acc.at[rows],
                    out_hbm.at[pl.ds(cid * HALF + sid * ROWS_PER_TILE,
                                     ROWS_PER_TILE)])


@functools.cache
def _sc_message_pass():
    mesh = plsc.VectorSubcoreMesh(core_axis_name="c", subcore_axis_name="s")
    return pl.kernel(
        _sc_body,
        mesh=mesh,
        out_type=jax.ShapeDtypeStruct((NP, H), _f32),
        scratch_types=[
            pltpu.VMEM((CHT, C), jnp.int32),
            pltpu.VMEM((CHT, C), jnp.int32),
            pltpu.VMEM((C,), jnp.int32),
            pltpu.VMEM((C,), jnp.int32),
            pltpu.VMEM((16,), jnp.int32),
            pltpu.VMEM((C, H), _f32),
            pltpu.VMEM_SHARED((HALF, H), _f32),
            pltpu.SemaphoreType.DMA,
        ],
    )


def kernel(x, edge_index, edge_attr, batch, W_emb, b_emb, edge_W,
           W_layers, b_layers, W_head, b_head):
    x_pad = jnp.zeros((NP, D_IN), _f32).at[:N].set(x)
    s0 = edge_index[0].astype(jnp.int32)
    d0 = edge_index[1].astype(jnp.int32)
    lo = (d0 < HALF).astype(jnp.int32)
    ar = jnp.arange(E, dtype=jnp.int32)
    cl = jnp.cumsum(lo)
    k0 = cl[E - 1]
    k1 = E - k0
    i1 = ar + 1
    lst0 = jnp.where(i1 <= k0,
                     jnp.searchsorted(cl, i1, side="left"), -1)
    lst1 = jnp.where(i1 <= k1,
                     jnp.searchsorted(i1 - cl, i1, side="left"), -1)

    def _core_lists(lst, dst_rebased):
        lst = lst.astype(jnp.int32).reshape(EPT, 16).T.reshape(16, CHT, C)
        valid = lst >= 0
        lc = jnp.clip(lst, 0)
        packed = lax.shift_left(dst_rebased[lc], 14) | s0[lc]
        return lst, jnp.where(valid, packed, -1)

    ei0, pk0 = _core_lists(lst0, d0)
    ei1, pk1 = _core_lists(lst1, d0 - HALF)
    ei = jnp.stack([ei0, ei1]).reshape(32, CHT, C)
    pk = jnp.stack([pk0, pk1]).reshape(32, CHT, C)
    sl16 = jnp.arange(16, dtype=jnp.int32)
    cnt = (jnp.maximum(jnp.stack([k0, k1])[:, None] - sl16, 0) + 15) // 16
    jmax = (cnt + C - 1) // C
    jm = jnp.broadcast_to(jmax.reshape(32, 1), (32, 16)).astype(jnp.int32)
    batch_col = jnp.full((NP, 1), G, jnp.int32).at[:N, 0].set(
        batch.astype(jnp.int32))
    b_emb2 = b_emb.reshape(1, H)
    b_head2 = b_head.reshape(1, T)

    h = _tc_embed(x_pad, W_emb, b_emb2)
    e = _tc_edge(edge_attr, edge_W)
    for l in range(L):
        agg = _sc_message_pass()(h, e, pk, ei, jm)
        h = _tc_update(agg, W_layers[l], b_layers[l].reshape(1, H), h)
    return _tc_pool_head(h, batch_col, W_head, b_head2)


# speedup vs baseline: 15.2330x; 15.2330x over previous
import functools

import jax
import jax.numpy as jnp
from jax import lax
from jax.experimental import pallas as pl
from jax.experimental.pallas import tpu as pltpu
from jax.experimental.pallas import tpu_sc as plsc

N, E, D_IN, H, L, T, G, DE = 10000, 320000, 128, 128, 3, 12, 64, 16
NP = 10240
HALF = NP // 2
EPT = E // 16
C = 80
CHT = EPT // C
ROWS_PER_TILE = HALF // 16

_f32 = jnp.float32


def _embed_body(x_ref, w_ref, b_ref, o_ref):
    o_ref[...] = jnp.dot(x_ref[...], w_ref[...],
                         preferred_element_type=_f32) + b_ref[...]


def _tc_embed(x_pad, W_emb, b_emb2):
    blk = NP // 10
    return pl.pallas_call(
        _embed_body,
        grid=(10,),
        in_specs=[
            pl.BlockSpec((blk, D_IN), lambda i: (i, 0)),
            pl.BlockSpec((D_IN, H), lambda i: (0, 0)),
            pl.BlockSpec((1, H), lambda i: (0, 0)),
        ],
        out_specs=pl.BlockSpec((blk, H), lambda i: (i, 0)),
        out_shape=jax.ShapeDtypeStruct((NP, H), _f32),
    )(x_pad, W_emb, b_emb2)


def _edge_body(a_ref, w_ref, o_ref):
    o_ref[...] = jnp.dot(a_ref[...], w_ref[...], preferred_element_type=_f32)


def _tc_edge(edge_attr, edge_W):
    blk = E // 80
    return pl.pallas_call(
        _edge_body,
        grid=(80,),
        in_specs=[
            pl.BlockSpec((blk, DE), lambda i: (i, 0)),
            pl.BlockSpec((DE, H), lambda i: (0, 0)),
        ],
        out_specs=pl.BlockSpec((blk, H), lambda i: (i, 0)),
        out_shape=jax.ShapeDtypeStruct((E, H), _f32),
    )(edge_attr, edge_W)


def _update_body(agg_ref, w_ref, b_ref, h_ref, o_ref):
    u = jnp.dot(agg_ref[...], w_ref[...], preferred_element_type=_f32) + b_ref[...]
    o_ref[...] = jnp.maximum(u, 0.0) + h_ref[...]


def _tc_update(agg, W, b2, h):
    blk = NP // 10
    return pl.pallas_call(
        _update_body,
        grid=(10,),
        in_specs=[
            pl.BlockSpec((blk, H), lambda i: (i, 0)),
            pl.BlockSpec((H, H), lambda i: (0, 0)),
            pl.BlockSpec((1, H), lambda i: (0, 0)),
            pl.BlockSpec((blk, H), lambda i: (i, 0)),
        ],
        out_specs=pl.BlockSpec((blk, H), lambda i: (i, 0)),
        out_shape=jax.ShapeDtypeStruct((NP, H), _f32),
    )(agg, W, b2, h)


def _pool_head_body(h_ref, batch_ref, wh_ref, bh_ref, o_ref):
    onehot = (batch_ref[...] ==
              lax.broadcasted_iota(jnp.int32, (NP, G), 1)).astype(_f32)
    graph_repr = lax.dot_general(onehot, h_ref[...],
                                 (((0,), (0,)), ((), ())),
                                 preferred_element_type=_f32)
    o_ref[...] = jnp.dot(graph_repr, wh_ref[...],
                         preferred_element_type=_f32) + bh_ref[...]


def _tc_pool_head(h, batch_col, W_head, b_head2):
    return pl.pallas_call(
        _pool_head_body,
        in_specs=[
            pl.BlockSpec((NP, H), lambda: (0, 0)),
            pl.BlockSpec((NP, 1), lambda: (0, 0)),
            pl.BlockSpec((H, T), lambda: (0, 0)),
            pl.BlockSpec((1, T), lambda: (0, 0)),
        ],
        out_specs=pl.BlockSpec((G, T), lambda: (0, 0)),
        out_shape=jax.ShapeDtypeStruct((G, T), _f32),
    )(h, batch_col, W_head, b_head2)


def _sc_body(h_hbm, e_hbm, src_hbm, dst_hbm, out_hbm,
             src_v, dst_v, ei_buf, e_buf, acc, sem):
    cid = lax.axis_index("c")
    sid = lax.axis_index("s")
    wid = cid * 16 + sid

    pltpu.sync_copy(src_hbm.at[wid], src_v)
    pltpu.sync_copy(dst_hbm.at[wid], dst_v)

    def zrow(r, carry):
        for q in range(H // 16):
            e_buf[r, pl.ds(q * 16, 16)] = jnp.zeros((16,), _f32)
        return carry

    lax.fori_loop(0, C, zrow, 0)
    for k in range(ROWS_PER_TILE // C):
        pltpu.sync_copy(e_buf, acc.at[pl.ds(sid * ROWS_PER_TILE + k * C, C)])

    plsc.subcore_barrier()

    def chunk(j, carry):
        base = sid * EPT + j * C
        for q in range(C // 16):
            sl = pl.ds(q * 16, 16)
            d = dst_v[j, sl]
            ramp = lax.broadcasted_iota(jnp.int32, (16,), 0) + (base + q * 16)
            ei_buf[sl] = jnp.where(d < 0, -1, ramp)
        pltpu.sync_copy(
            e_hbm.at[plsc.Indices(ei_buf, ignored_value=-1)], e_buf)
        pltpu.sync_copy(
            h_hbm.at[plsc.Indices(src_v.at[j], ignored_value=-1)],
            e_buf, add=True)

        def row(r, carry2):
            for q in range(H // 16):
                sl = pl.ds(q * 16, 16)
                e_buf[r, sl] = jnp.maximum(e_buf[r, sl], 0.0)
            return carry2

        lax.fori_loop(0, C, row, 0)
        pltpu.sync_copy(
            e_buf, acc.at[plsc.Indices(dst_v.at[j], ignored_value=-1)],
            add=True)
        return carry

    lax.fori_loop(0, CHT, chunk, 0)
    plsc.subcore_barrier()
    rows = pl.ds(sid * ROWS_PER_TILE, ROWS_PER_TILE)
    pltpu.sync_copy(acc.at[rows],
                    out_hbm.at[pl.ds(cid * HALF + sid * ROWS_PER_TILE,
                                     ROWS_PER_TILE)])


@functools.cache
def _sc_message_pass():
    mesh = plsc.VectorSubcoreMesh(core_axis_name="c", subcore_axis_name="s")
    return pl.kernel(
        _sc_body,
        mesh=mesh,
        out_type=jax.ShapeDtypeStruct((NP, H), _f32),
        scratch_types=[
            pltpu.VMEM((CHT, C), jnp.int32),
            pltpu.VMEM((CHT, C), jnp.int32),
            pltpu.VMEM((C,), jnp.int32),
            pltpu.VMEM((C, H), _f32),
            pltpu.VMEM_SHARED((HALF, H), _f32),
            pltpu.SemaphoreType.DMA,
        ],
    )


def kernel(x, edge_index, edge_attr, batch, W_emb, b_emb, edge_W,
           W_layers, b_layers, W_head, b_head):
    x_pad = jnp.zeros((NP, D_IN), _f32).at[:N].set(x)
    s0 = edge_index[0].astype(jnp.int32)
    d0 = edge_index[1].astype(jnp.int32)
    lo = d0 < HALF
    src = jnp.stack([jnp.where(lo, s0, -1),
                     jnp.where(lo, -1, s0)]).reshape(32, CHT, C)
    dst = jnp.stack([jnp.where(lo, d0, -1),
                     jnp.where(lo, -1, d0 - HALF)]).reshape(32, CHT, C)
    batch_col = jnp.full((NP, 1), G, jnp.int32).at[:N, 0].set(
        batch.astype(jnp.int32))
    b_emb2 = b_emb.reshape(1, H)
    b_head2 = b_head.reshape(1, T)

    h = _tc_embed(x_pad, W_emb, b_emb2)
    e = _tc_edge(edge_attr, edge_W)
    for l in range(L):
        agg = _sc_message_pass()(h, e, src, dst)
        h = _tc_update(agg, W_layers[l], b_layers[l].reshape(1, H), h)
    return _tc_pool_head(h, batch_col, W_head, b_head2)
